# baseline (speedup 1.0000x reference)
import functools

import jax
import jax.numpy as jnp
from jax import lax
from jax.experimental import pallas as pl
from jax.experimental.pallas import tpu as pltpu
from jax.experimental.pallas import tpu_sc as plsc

V = 1000000
D = 32
B = 16384
NC = 2
NS = 16
L = 16
NW = NC * NS
BPW = B // NW
NGRP = BPW // L


def _rsqrt(x):
    i = plsc.bitcast(x, jnp.int32)
    i = jnp.int32(0x5F3759DF) - (i >> 1)
    y = plsc.bitcast(i, jnp.float32)
    for _ in range(3):
        y = y * (1.5 - 0.5 * x * y * y)
    return y


def _sc_body(src_hbm, anc_hbm, rb_hbm, emb1_hbm, emb2_hbm, out_hbm,
             sidx_v, aidx_v, rb_v, g1_v, g2_v, h1_v, h2_v, out_v, sem):
    wid = lax.axis_index("s") * NC + lax.axis_index("c")
    base = wid * BPW

    pltpu.sync_copy(src_hbm.at[pl.ds(base, BPW)], sidx_v)
    pltpu.sync_copy(anc_hbm.at[pl.ds(base, BPW)], aidx_v)
    pltpu.sync_copy(rb_hbm, rb_v)

    c1 = pltpu.async_copy(emb1_hbm.at[sidx_v], g1_v, sem)
    c2 = pltpu.async_copy(emb2_hbm.at[sidx_v], g2_v, sem)
    c3 = pltpu.async_copy(emb1_hbm.at[aidx_v], h1_v, sem)
    c4 = pltpu.async_copy(emb2_hbm.at[aidx_v], h2_v, sem)
    c1.wait()
    c2.wait()
    c3.wait()
    c4.wait()

    iota = lax.iota(jnp.int32, L)
    zero = jnp.zeros((L,), jnp.float32)

    def group(g, carry):
        rows = g * L + iota
        d1 = d2 = d12 = n1 = n2 = n12 = m1 = m2 = zero
        for d in range(D):
            col = jnp.full((L,), d, jnp.int32)
            g1d = plsc.load_gather(g1_v, [rows, col])
            g2d = plsc.load_gather(g2_v, [rows, col])
            h1d = plsc.load_gather(h1_v, [rows, col])
            h2d = plsc.load_gather(h2_v, [rows, col])
            r1d = rb_v[d]
            r2d = rb_v[D + d]
            r12d = rb_v[2 * D + d]
            x1 = g1d * r1d
            x2 = g2d * r2d
            x12 = g1d * r12d * g2d * h1d
            d1 = d1 + x1 * h1d
            n1 = n1 + x1 * x1
            m1 = m1 + h1d * h1d
            d2 = d2 + x2 * h2d
            n2 = n2 + x2 * x2
            m2 = m2 + h2d * h2d
            d12 = d12 + x12 * h2d
            n12 = n12 + x12 * x12
        res = (d1 * _rsqrt(jnp.maximum(n1 * m1, 1e-24))
               + d2 * _rsqrt(jnp.maximum(n2 * m2, 1e-24))
               - d12 * _rsqrt(jnp.maximum(n12 * m2, 1e-24)))
        out_v[pl.ds(g * L, L)] = res
        return carry

    lax.fori_loop(0, NGRP, group, 0)
    pltpu.sync_copy(out_v, out_hbm.at[pl.ds(base, BPW)])


_sc_call = pl.kernel(
    _sc_body,
    out_type=jax.ShapeDtypeStruct((B,), jnp.float32),
    mesh=plsc.VectorSubcoreMesh(core_axis_name="c", subcore_axis_name="s",
                                num_cores=NC, num_subcores=NS),
    compiler_params=pltpu.CompilerParams(needs_layout_passes=False,
                                         use_tc_tiling_on_sc=False),
    scratch_types=[
        pltpu.VMEM((BPW,), jnp.int32),
        pltpu.VMEM((BPW,), jnp.int32),
        pltpu.VMEM((3 * D, L), jnp.float32),
        pltpu.VMEM((BPW, D), jnp.float32),
        pltpu.VMEM((BPW, D), jnp.float32),
        pltpu.VMEM((BPW, D), jnp.float32),
        pltpu.VMEM((BPW, D), jnp.float32),
        pltpu.VMEM((BPW,), jnp.float32),
        pltpu.SemaphoreType.DMA,
    ],
)


def kernel(source_nodes, anchor_nodes, rel_id, emb1, emb2, rvecs1, rvecs2):
    src = source_nodes.astype(jnp.int32)
    anc = anchor_nodes.astype(jnp.int32)
    r1 = rvecs1[rel_id]
    r2 = rvecs2[rel_id]
    rb = jnp.concatenate([
        jnp.broadcast_to(r1[:, None], (D, L)),
        jnp.broadcast_to(r2[:, None], (D, L)),
        jnp.broadcast_to((r1 * r2)[:, None], (D, L)),
    ], axis=0)
    return _sc_call(src, anc, rb, emb1, emb2)
